# baseline (speedup 1.0000x reference)
import functools

import jax
import jax.numpy as jnp
from jax import lax
from jax.experimental import pallas as pl
from jax.experimental.pallas import tpu as pltpu
from jax.experimental.pallas import tpu_sc as plsc

V = 100000
E = 256
H = 512
D = 512
B = 128
L = 128
B3 = 3 * B
N_TOK = L * B3
NSPLIT = 2
LS = L // NSPLIT


def _gather_sc(table, idx, n_tok):
    info = plsc.get_sparse_core_info()
    nw = info.num_cores * info.num_subcores
    bpw = n_tok // nw
    ch = 128
    nchunk = bpw // ch
    mesh = plsc.VectorSubcoreMesh(core_axis_name="c", subcore_axis_name="s")

    @functools.partial(
        pl.kernel,
        mesh=mesh,
        out_type=jax.ShapeDtypeStruct((n_tok, E), jnp.float32),
        scratch_types=[
            pltpu.VMEM((bpw,), jnp.int32),
            pltpu.VMEM((ch, E), jnp.float32),
            pltpu.VMEM((ch, E), jnp.float32),
            pltpu.SemaphoreType.DMA,
            pltpu.SemaphoreType.DMA,
            pltpu.SemaphoreType.DMA,
            pltpu.SemaphoreType.DMA,
        ],
    )
    def k(table_hbm, idx_hbm, out_hbm, idx_v, rows0, rows1,
          gsem0, gsem1, wsem0, wsem1):
        cid = lax.axis_index("c")
        sid = lax.axis_index("s")
        wid = sid * info.num_cores + cid
        base = wid * bpw
        pltpu.sync_copy(idx_hbm.at[pl.ds(base, bpw)], idx_v)

        bufs = (rows0, rows1)
        gsems = (gsem0, gsem1)
        wsems = (wsem0, wsem1)

        def grab(i, b):
            return pltpu.async_copy(
                table_hbm.at[idx_v.at[pl.ds(i * ch, ch)]], bufs[b], gsems[b])

        gh = [grab(0, 0), grab(1, 1)]
        wh = [None, None]
        for i in range(nchunk):
            b = i & 1
            gh[b].wait()
            wh[b] = pltpu.async_copy(
                bufs[b], out_hbm.at[pl.ds(base + i * ch, ch)], wsems[b])
            if i + 2 < nchunk:
                wh[b].wait()
                gh[b] = grab(i + 2, b)
        wh[0].wait()
        wh[1].wait()

    return k(table, idx)


TU = 8


def _make_lstm_body(nsteps, final):
    ngrid = nsteps // TU

    def body(x_ref, W_ref, U_ref, b_ref, W1_ref, b1_ref, W2_ref,
             b2_ref, hin_ref, cin_ref, *out_and_scr):
        if final:
            out_ref, h_scr, c_scr = out_and_scr
        else:
            hout_ref, cout_ref, h_scr, c_scr = out_and_scr
        t = pl.program_id(0)

        @pl.when(t == 0)
        def _init():
            h_scr[...] = hin_ref[...]
            c_scr[...] = cin_ref[...]

        def sig(v):
            return 0.5 * jnp.tanh(0.5 * v) + 0.5

        h = h_scr[...]
        c = c_scr[...]
        xt_all = x_ref[...].reshape(TU * B3, E).astype(jnp.bfloat16)
        zx = [jnp.dot(xt_all, W_ref[:, k * H:(k + 1) * H],
                      preferred_element_type=jnp.float32)
              + b_ref[:, k * H:(k + 1) * H] for k in range(4)]
        h_new = None
        for s in range(TU):
            def zchunk(k, h=h):
                return (zx[k][s * B3:(s + 1) * B3]
                        + jnp.dot(h, U_ref[:, k * H:(k + 1) * H],
                                  preferred_element_type=jnp.float32))

            i = sig(zchunk(0))
            f = sig(zchunk(1))
            g = jnp.tanh(zchunk(2))
            o = sig(zchunk(3))
            c = f * c + i * g
            h_new = o * jnp.tanh(c)
            h = h_new.astype(jnp.bfloat16)
        c_scr[...] = c
        h_scr[...] = h

        @pl.when(t == ngrid - 1)
        def _fin():
            if final:
                h_a = h_new[:B]
                h_p = h_new[B:2 * B]
                h_n = h_new[2 * B:]
                hcat = jnp.concatenate(
                    [jnp.concatenate([h_a, h_p], axis=1),
                     jnp.concatenate([h_a, h_n], axis=1)], axis=0)
                h1 = jnp.maximum(
                    jnp.dot(hcat, W1_ref[...],
                            preferred_element_type=jnp.float32)
                    + b1_ref[...], 0.0)
                s_out = jax.nn.sigmoid(
                    jnp.dot(h1, W2_ref[...],
                            preferred_element_type=jnp.float32)
                    + b2_ref[0, 0])
                out_ref[...] = jnp.concatenate(
                    [s_out[:B], s_out[B:]], axis=1)
            else:
                hout_ref[...] = h
                cout_ref[...] = c

    return body, ngrid


def _lstm_tc_part(x, W, U, b, W1, b1, W2, b2, h_in, c_in, final):
    nsteps = x.shape[0]
    body, ngrid = _make_lstm_body(nsteps, final)
    full = lambda shape: pl.BlockSpec(shape, lambda t: (0,) * len(shape))
    if final:
        out_shape = jax.ShapeDtypeStruct((B, 2), jnp.float32)
        out_specs = pl.BlockSpec((B, 2), lambda t: (0, 0))
    else:
        out_shape = (jax.ShapeDtypeStruct((B3, H), jnp.bfloat16),
                     jax.ShapeDtypeStruct((B3, H), jnp.float32))
        out_specs = (full((B3, H)), full((B3, H)))
    return pl.pallas_call(
        body,
        grid=(ngrid,),
        in_specs=[
            pl.BlockSpec((TU, B3, E), lambda t: (t, 0, 0)),
            full((E, 4 * H)),
            full((H, 4 * H)),
            full((1, 4 * H)),
            full((2 * H, D)),
            full((1, D)),
            full((D, 1)),
            pl.BlockSpec(memory_space=pltpu.SMEM),
            full((B3, H)),
            full((B3, H)),
        ],
        out_specs=out_specs,
        out_shape=out_shape,
        scratch_shapes=[
            pltpu.VMEM((B3, H), jnp.bfloat16),
            pltpu.VMEM((B3, H), jnp.float32),
        ],
        compiler_params=pltpu.CompilerParams(
            dimension_semantics=("arbitrary",)),
    )(x, W, U, b, W1, b1, W2, b2, h_in, c_in)


def kernel(anchor_tokens, pos_tokens, neg_tokens, emb_table, W, U, b,
           W1, b1, W2, b2):
    toks = jnp.stack([anchor_tokens.astype(jnp.int32),
                      pos_tokens.astype(jnp.int32),
                      neg_tokens.astype(jnp.int32)])
    idx = toks.reshape(B3, L).T.reshape(-1)
    part = N_TOK // NSPLIT
    xs = [_gather_sc(emb_table, idx[k * part:(k + 1) * part],
                     part).reshape(LS, B3, E) for k in range(NSPLIT)]
    Wb = W.astype(jnp.bfloat16)
    Ub = U.astype(jnp.bfloat16)
    b2d = b.reshape(1, 4 * H)
    b1d = b1.reshape(1, D)
    b2s = b2.reshape(1, 1)
    h = jnp.zeros((B3, H), jnp.bfloat16)
    c = jnp.zeros((B3, H), jnp.float32)
    for k in range(NSPLIT - 1):
        h, c = _lstm_tc_part(xs[k], Wb, Ub, b2d, W1, b1d, W2, b2s, h, c,
                             final=False)
    return _lstm_tc_part(xs[-1], Wb, Ub, b2d, W1, b1d, W2, b2s, h, c,
                         final=True)


# speedup vs baseline: 1.1075x; 1.1075x over previous
import functools

import jax
import jax.numpy as jnp
from jax import lax
from jax.experimental import pallas as pl
from jax.experimental.pallas import tpu as pltpu
from jax.experimental.pallas import tpu_sc as plsc

V = 100000
E = 256
H = 512
D = 512
B = 128
L = 128
B3 = 3 * B
N_TOK = L * B3
NSPLIT = 2
LS = L // NSPLIT


def _gather_sc(table, idx, n_tok):
    info = plsc.get_sparse_core_info()
    nw = info.num_cores * info.num_subcores
    bpw = n_tok // nw
    ch = 128
    nchunk = bpw // ch
    mesh = plsc.VectorSubcoreMesh(core_axis_name="c", subcore_axis_name="s")

    @functools.partial(
        pl.kernel,
        mesh=mesh,
        out_type=jax.ShapeDtypeStruct((n_tok, E), jnp.float32),
        scratch_types=[
            pltpu.VMEM((bpw,), jnp.int32),
            pltpu.VMEM((ch, E), jnp.float32),
            pltpu.VMEM((ch, E), jnp.float32),
            pltpu.SemaphoreType.DMA,
            pltpu.SemaphoreType.DMA,
            pltpu.SemaphoreType.DMA,
            pltpu.SemaphoreType.DMA,
        ],
    )
    def k(table_hbm, idx_hbm, out_hbm, idx_v, rows0, rows1,
          gsem0, gsem1, wsem0, wsem1):
        cid = lax.axis_index("c")
        sid = lax.axis_index("s")
        wid = sid * info.num_cores + cid
        base = wid * bpw
        pltpu.sync_copy(idx_hbm.at[pl.ds(base, bpw)], idx_v)

        bufs = (rows0, rows1)
        gsems = (gsem0, gsem1)
        wsems = (wsem0, wsem1)

        def grab(i, b):
            return pltpu.async_copy(
                table_hbm.at[idx_v.at[pl.ds(i * ch, ch)]], bufs[b], gsems[b])

        gh = [grab(0, 0), grab(1, 1)]
        wh = [None, None]
        for i in range(nchunk):
            b = i & 1
            gh[b].wait()
            wh[b] = pltpu.async_copy(
                bufs[b], out_hbm.at[pl.ds(base + i * ch, ch)], wsems[b])
            if i + 2 < nchunk:
                wh[b].wait()
                gh[b] = grab(i + 2, b)
        wh[0].wait()
        wh[1].wait()

    return k(table, idx)


TU = 8


def _make_lstm_body(nsteps, final):
    ngrid = nsteps // TU

    def body(x_ref, W_ref, U_ref, b_ref, W1_ref, b1_ref, W2_ref,
             b2_ref, hin_ref, cin_ref, *out_and_scr):
        if final:
            out_ref, h_scr, c_scr = out_and_scr
        else:
            hout_ref, cout_ref, h_scr, c_scr = out_and_scr
        t = pl.program_id(0)

        @pl.when(t == 0)
        def _init():
            h_scr[...] = hin_ref[...]
            c_scr[...] = cin_ref[...]

        def sig(v):
            return 0.5 * jnp.tanh(0.5 * v) + 0.5

        h = h_scr[...]
        c = c_scr[...]
        h_new = None
        for s in range(TU):
            xt = x_ref[s].astype(jnp.bfloat16)

            def zchunk(k, h=h, xt=xt):
                return (jnp.dot(xt, W_ref[:, k * H:(k + 1) * H],
                                preferred_element_type=jnp.float32)
                        + jnp.dot(h, U_ref[:, k * H:(k + 1) * H],
                                  preferred_element_type=jnp.float32)
                        + b_ref[:, k * H:(k + 1) * H])

            i = sig(zchunk(0))
            f = sig(zchunk(1))
            g = jnp.tanh(zchunk(2))
            o = sig(zchunk(3))
            c = f * c + i * g
            h_new = o * jnp.tanh(c)
            h = h_new.astype(jnp.bfloat16)
        c_scr[...] = c
        h_scr[...] = h

        @pl.when(t == ngrid - 1)
        def _fin():
            if final:
                h_a = h_new[:B]
                h_p = h_new[B:2 * B]
                h_n = h_new[2 * B:]
                hcat = jnp.concatenate(
                    [jnp.concatenate([h_a, h_p], axis=1),
                     jnp.concatenate([h_a, h_n], axis=1)], axis=0)
                h1 = jnp.maximum(
                    jnp.dot(hcat, W1_ref[...],
                            preferred_element_type=jnp.float32)
                    + b1_ref[...], 0.0)
                s_out = jax.nn.sigmoid(
                    jnp.dot(h1, W2_ref[...],
                            preferred_element_type=jnp.float32)
                    + b2_ref[0, 0])
                out_ref[...] = jnp.concatenate(
                    [s_out[:B], s_out[B:]], axis=1)
            else:
                hout_ref[...] = h
                cout_ref[...] = c

    return body, ngrid


def _lstm_tc_part(x, W, U, b, W1, b1, W2, b2, h_in, c_in, final):
    nsteps = x.shape[0]
    body, ngrid = _make_lstm_body(nsteps, final)
    full = lambda shape: pl.BlockSpec(shape, lambda t: (0,) * len(shape))
    if final:
        out_shape = jax.ShapeDtypeStruct((B, 2), jnp.float32)
        out_specs = pl.BlockSpec((B, 2), lambda t: (0, 0))
    else:
        out_shape = (jax.ShapeDtypeStruct((B3, H), jnp.bfloat16),
                     jax.ShapeDtypeStruct((B3, H), jnp.float32))
        out_specs = (full((B3, H)), full((B3, H)))
    return pl.pallas_call(
        body,
        grid=(ngrid,),
        in_specs=[
            pl.BlockSpec((TU, B3, E), lambda t: (t, 0, 0)),
            full((E, 4 * H)),
            full((H, 4 * H)),
            full((1, 4 * H)),
            full((2 * H, D)),
            full((1, D)),
            full((D, 1)),
            pl.BlockSpec(memory_space=pltpu.SMEM),
            full((B3, H)),
            full((B3, H)),
        ],
        out_specs=out_specs,
        out_shape=out_shape,
        scratch_shapes=[
            pltpu.VMEM((B3, H), jnp.bfloat16),
            pltpu.VMEM((B3, H), jnp.float32),
        ],
        compiler_params=pltpu.CompilerParams(
            dimension_semantics=("arbitrary",)),
    )(x, W, U, b, W1, b1, W2, b2, h_in, c_in)


def kernel(anchor_tokens, pos_tokens, neg_tokens, emb_table, W, U, b,
           W1, b1, W2, b2):
    toks = jnp.stack([anchor_tokens.astype(jnp.int32),
                      pos_tokens.astype(jnp.int32),
                      neg_tokens.astype(jnp.int32)])
    idx = toks.reshape(B3, L).T.reshape(-1)
    part = N_TOK // NSPLIT
    xs = [_gather_sc(emb_table, idx[k * part:(k + 1) * part],
                     part).reshape(LS, B3, E) for k in range(NSPLIT)]
    Wb = W.astype(jnp.bfloat16)
    Ub = U.astype(jnp.bfloat16)
    b2d = b.reshape(1, 4 * H)
    b1d = b1.reshape(1, D)
    b2s = b2.reshape(1, 1)
    h = jnp.zeros((B3, H), jnp.bfloat16)
    c = jnp.zeros((B3, H), jnp.float32)
    for k in range(NSPLIT - 1):
        h, c = _lstm_tc_part(xs[k], Wb, Ub, b2d, W1, b1d, W2, b2s, h, c,
                             final=False)
    return _lstm_tc_part(xs[-1], Wb, Ub, b2d, W1, b1d, W2, b2s, h, c,
                         final=True)


# speedup vs baseline: 1.1525x; 1.0406x over previous
import functools

import jax
import jax.numpy as jnp
from jax import lax
from jax.experimental import pallas as pl
from jax.experimental.pallas import tpu as pltpu
from jax.experimental.pallas import tpu_sc as plsc

V = 100000
E = 256
H = 512
D = 512
B = 128
L = 128
B3 = 3 * B
N_TOK = L * B3
NSPLIT = 2
LS = L // NSPLIT


def _gather_sc(table, idx, n_tok):
    info = plsc.get_sparse_core_info()
    nw = info.num_cores * info.num_subcores
    bpw = n_tok // nw
    ch = 128
    nchunk = bpw // ch
    mesh = plsc.VectorSubcoreMesh(core_axis_name="c", subcore_axis_name="s")

    @functools.partial(
        pl.kernel,
        mesh=mesh,
        out_type=jax.ShapeDtypeStruct((n_tok, E), jnp.float32),
        scratch_types=[
            pltpu.VMEM((bpw,), jnp.int32),
            pltpu.VMEM((ch, E), jnp.float32),
            pltpu.VMEM((ch, E), jnp.float32),
            pltpu.SemaphoreType.DMA,
            pltpu.SemaphoreType.DMA,
            pltpu.SemaphoreType.DMA,
            pltpu.SemaphoreType.DMA,
        ],
    )
    def k(table_hbm, idx_hbm, out_hbm, idx_v, rows0, rows1,
          gsem0, gsem1, wsem0, wsem1):
        cid = lax.axis_index("c")
        sid = lax.axis_index("s")
        wid = sid * info.num_cores + cid
        base = wid * bpw
        pltpu.sync_copy(idx_hbm.at[pl.ds(base, bpw)], idx_v)

        bufs = (rows0, rows1)
        gsems = (gsem0, gsem1)
        wsems = (wsem0, wsem1)

        def grab(i, b):
            return pltpu.async_copy(
                table_hbm.at[idx_v.at[pl.ds(i * ch, ch)]], bufs[b], gsems[b])

        gh = [grab(0, 0), grab(1, 1)]
        wh = [None, None]
        for i in range(nchunk):
            b = i & 1
            gh[b].wait()
            wh[b] = pltpu.async_copy(
                bufs[b], out_hbm.at[pl.ds(base + i * ch, ch)], wsems[b])
            if i + 2 < nchunk:
                wh[b].wait()
                gh[b] = grab(i + 2, b)
        wh[0].wait()
        wh[1].wait()

    return k(table, idx)


TU = 8


def _make_lstm_body(nsteps, final):
    ngrid = nsteps // TU

    def body(x_ref, W_ref, U_ref, b_ref, W1_ref, b1_ref, W2_ref,
             b2_ref, hin_ref, cin_ref, *out_and_scr):
        if final:
            out_ref, h_scr, c_scr = out_and_scr
        else:
            hout_ref, cout_ref, h_scr, c_scr = out_and_scr
        t = pl.program_id(0)

        @pl.when(t == 0)
        def _init():
            h_scr[...] = hin_ref[...]
            c_scr[...] = cin_ref[...]

        def sig(v):
            return 0.5 * jnp.tanh(0.5 * v) + 0.5

        h = h_scr[...]
        c = c_scr[...]
        h_new = None
        for s in range(TU):
            xt = x_ref[s].astype(jnp.bfloat16)

            def zchunk(k, h=h, xt=xt):
                return (jnp.dot(xt, W_ref[:, k * H:(k + 1) * H],
                                preferred_element_type=jnp.float32)
                        + jnp.dot(h, U_ref[:, k * H:(k + 1) * H],
                                  preferred_element_type=jnp.float32)
                        + b_ref[:, k * H:(k + 1) * H])

            i = sig(zchunk(0))
            f = sig(zchunk(1))
            g = jnp.tanh(zchunk(2))
            o = sig(zchunk(3))
            c = f * c + i * g
            h_new = o * jnp.tanh(c)
            h = h_new.astype(jnp.bfloat16)
        c_scr[...] = c
        h_scr[...] = h

        @pl.when(t == ngrid - 1)
        def _fin():
            if final:
                h_a = h_new[:B]
                h_p = h_new[B:2 * B]
                h_n = h_new[2 * B:]
                hcat = jnp.concatenate(
                    [jnp.concatenate([h_a, h_p], axis=1),
                     jnp.concatenate([h_a, h_n], axis=1)], axis=0)
                h1 = jnp.maximum(
                    jnp.dot(hcat, W1_ref[...],
                            preferred_element_type=jnp.float32)
                    + b1_ref[...], 0.0)
                s_out = jax.nn.sigmoid(
                    jnp.dot(h1, W2_ref[...],
                            preferred_element_type=jnp.float32)
                    + b2_ref[0, 0])
                out_ref[...] = jnp.concatenate(
                    [s_out[:B], s_out[B:]], axis=1)
            else:
                hout_ref[...] = h
                cout_ref[...] = c

    return body, ngrid


def _lstm_tc_part(x, W, U, b, W1, b1, W2, b2, h_in, c_in, final):
    nsteps = x.shape[0]
    body, ngrid = _make_lstm_body(nsteps, final)
    full = lambda shape: pl.BlockSpec(shape, lambda t: (0,) * len(shape))
    if final:
        out_shape = jax.ShapeDtypeStruct((B, 2), jnp.float32)
        out_specs = pl.BlockSpec((B, 2), lambda t: (0, 0))
    else:
        out_shape = (jax.ShapeDtypeStruct((B3, H), jnp.bfloat16),
                     jax.ShapeDtypeStruct((B3, H), jnp.float32))
        out_specs = (full((B3, H)), full((B3, H)))
    return pl.pallas_call(
        body,
        grid=(ngrid,),
        in_specs=[
            pl.BlockSpec((TU, B3, E), lambda t: (t, 0, 0)),
            full((E, 4 * H)),
            full((H, 4 * H)),
            full((1, 4 * H)),
            full((2 * H, D)),
            full((1, D)),
            full((D, 1)),
            pl.BlockSpec(memory_space=pltpu.SMEM),
            full((B3, H)),
            full((B3, H)),
        ],
        out_specs=out_specs,
        out_shape=out_shape,
        scratch_shapes=[
            pltpu.VMEM((B3, H), jnp.bfloat16),
            pltpu.VMEM((B3, H), jnp.float32),
        ],
        compiler_params=pltpu.CompilerParams(
            dimension_semantics=("arbitrary",)),
    )(x, W, U, b, W1, b1, W2, b2, h_in, c_in)


def kernel(anchor_tokens, pos_tokens, neg_tokens, emb_table, W, U, b,
           W1, b1, W2, b2):
    toks = jnp.stack([anchor_tokens.astype(jnp.int32),
                      pos_tokens.astype(jnp.int32),
                      neg_tokens.astype(jnp.int32)])
    idx = toks.reshape(B3, L).T.reshape(-1)
    steps = (32, 96)
    xs = []
    off = 0
    for ns in steps:
        n_tok = ns * B3
        xs.append(_gather_sc(emb_table, idx[off:off + n_tok],
                             n_tok).reshape(ns, B3, E))
        off += n_tok
    Wb = W.astype(jnp.bfloat16)
    Ub = U.astype(jnp.bfloat16)
    b2d = b.reshape(1, 4 * H)
    b1d = b1.reshape(1, D)
    b2s = b2.reshape(1, 1)
    h = jnp.zeros((B3, H), jnp.bfloat16)
    c = jnp.zeros((B3, H), jnp.float32)
    for xk in xs[:-1]:
        h, c = _lstm_tc_part(xk, Wb, Ub, b2d, W1, b1d, W2, b2s, h, c,
                             final=False)
    return _lstm_tc_part(xs[-1], Wb, Ub, b2d, W1, b1d, W2, b2s, h, c,
                         final=True)
